# baseline (speedup 1.0000x reference)
import functools

import jax
import jax.numpy as jnp
from jax import lax
from jax.experimental import pallas as pl
from jax.experimental.pallas import tpu as pltpu
from jax.experimental.pallas import tpu_sc as plsc

_EPS = 1e-5
_K = 32
_F32 = jnp.float32


def _fold(p):
    s = p['g'] / jnp.sqrt(1.0 + _EPS)
    W = p['W'] * s[:, None]
    b = p['b'] * s + p['bb']
    return W.T, b[None, :]


def _pad_cols(x, to):
    c = x.shape[-1]
    if c == to:
        return x
    pad = [(0, 0)] * (x.ndim - 1) + [(0, to - c)]
    return jnp.pad(x, pad)


def _spfe(xyz_pad, Wc, Wm, b):
    B, N, _ = xyz_pad.shape
    Cout = Wc.shape[1]

    def body(x_ref, wc_ref, wm_ref, b_ref, o_ref):
        x = x_ref[0]
        m = jnp.mean(x, axis=0, keepdims=True)
        y = (jnp.dot(x, wc_ref[...], preferred_element_type=_F32)
             - jnp.dot(m, wm_ref[...], preferred_element_type=_F32)
             + b_ref[...])
        o_ref[0] = jnp.maximum(y, 0.0)

    return pl.pallas_call(
        body,
        grid=(B,),
        in_specs=[
            pl.BlockSpec((1, N, 8), lambda i: (i, 0, 0)),
            pl.BlockSpec((8, Cout), lambda i: (0, 0)),
            pl.BlockSpec((8, Cout), lambda i: (0, 0)),
            pl.BlockSpec((1, Cout), lambda i: (0, 0)),
        ],
        out_specs=pl.BlockSpec((1, N, Cout), lambda i: (i, 0, 0)),
        out_shape=jax.ShapeDtypeStruct((B, N, Cout), _F32),
    )(xyz_pad, Wc, Wm, b)


def _knn(centersT_pad, xyz_pad, k, Mt):
    B = centersT_pad.shape[0]
    M = centersT_pad.shape[2]
    N = xyz_pad.shape[1]

    cw = 128
    nc = N // cw
    T = 8 if nc >= 32 else 16
    chunked = nc >= 8
    Csz = nc * T
    INF = float('inf')

    def body(p_ref, c_ref, idx_ref, d_ref, cv_ref, ci_ref):
        b = pl.program_id(0)
        p = p_ref[0]
        cT = c_ref[0]
        pp = jnp.sum(p * p, axis=1, keepdims=True)
        cc = jnp.sum(cT * cT, axis=0, keepdims=True)
        d2 = pp + cc - 2.0 * jnp.dot(p, cT, preferred_element_type=_F32)
        d_ref[...] = jnp.sqrt(jnp.maximum(d2, 0.0))
        sub = lax.broadcasted_iota(jnp.int32, (N, Mt), 0)
        krow = lax.broadcasted_iota(jnp.int32, (k, Mt), 0)

        def full_extract():
            def it(i, acc):
                d_ = d_ref[...]
                mv = jnp.min(d_, axis=0, keepdims=True)
                sel = jnp.min(jnp.where(d_ <= mv, sub, N), axis=0,
                              keepdims=True)
                acc = jnp.where(krow == i, sel, acc)
                d_ref[...] = jnp.where(sub == sel, INF, d_)
                return acc

            return lax.fori_loop(0, k, it, jnp.zeros((k, Mt), jnp.int32))

        if not chunked:
            idx_ref[0] = full_extract() + b * N
            return

        subw = lax.broadcasted_iota(jnp.int32, (cw, Mt), 0)
        trow = lax.broadcasted_iota(jnp.int32, (T, Mt), 0)

        def per_chunk(c, carry):
            slab = d_ref[pl.ds(c * cw, cw), :]
            cand_v = jnp.full((T, Mt), INF, _F32)
            cand_i = jnp.zeros((T, Mt), jnp.int32)
            for t in range(T):
                mv = jnp.min(slab, axis=0, keepdims=True)
                sel = jnp.min(jnp.where(slab <= mv, subw, cw), axis=0,
                              keepdims=True)
                cand_v = jnp.where(trow == t, mv, cand_v)
                cand_i = jnp.where(trow == t, sel + c * cw, cand_i)
                slab = jnp.where(subw == sel, INF, slab)
            cv_ref[pl.ds(c * T, T), :] = cand_v
            ci_ref[pl.ds(c * T, T), :] = cand_i
            return carry

        lax.fori_loop(0, nc, per_chunk, 0)

        sub5 = lax.broadcasted_iota(jnp.int32, (Csz, Mt), 0)

        def rnd(i, carry):
            acc, bad = carry
            cv = cv_ref[...]
            mv = jnp.min(cv, axis=0, keepdims=True)
            sel = jnp.min(jnp.where(cv <= mv, sub5, Csz), axis=0,
                          keepdims=True)
            hit = sub5 == sel
            idxsel = jnp.min(jnp.where(hit, ci_ref[...],
                                       jnp.int32(0x7FFFFFFF)),
                             axis=0, keepdims=True)
            acc = jnp.where(krow == i, idxsel, acc)
            bad = jnp.maximum(bad, (sel % T == T - 1).astype(jnp.int32))
            cv_ref[...] = jnp.where(hit, INF, cv)
            return acc, bad

        acc, bad = lax.fori_loop(
            0, k, rnd,
            (jnp.zeros((k, Mt), jnp.int32), jnp.zeros((1, Mt), jnp.int32)))
        idx_ref[0] = acc + b * N

        @pl.when(jnp.max(bad) > 0)
        def _():
            idx_ref[0] = full_extract() + b * N

    return pl.pallas_call(
        body,
        grid=(B, M // Mt),
        in_specs=[
            pl.BlockSpec((1, N, 8), lambda b, m: (b, 0, 0)),
            pl.BlockSpec((1, 8, Mt), lambda b, m: (b, 0, m)),
        ],
        out_specs=pl.BlockSpec((1, k, Mt), lambda b, m: (b, 0, m)),
        out_shape=jax.ShapeDtypeStruct((B, k, M), jnp.int32),
        scratch_shapes=[
            pltpu.VMEM((N, Mt), _F32),
            pltpu.VMEM((max(Csz, 8), Mt), _F32),
            pltpu.VMEM((max(Csz, 8), Mt), jnp.int32),
        ],
    )(xyz_pad, centersT_pad)


def _sc_gather(table, idx):
    R, Dp = table.shape
    (Btot,) = idx.shape
    info = plsc.get_sparse_core_info()
    NW = info.num_cores * info.num_subcores
    CH = 128
    b_per_w = Btot // NW
    nch = b_per_w // CH
    mesh = plsc.VectorSubcoreMesh(core_axis_name="c", subcore_axis_name="s")

    @functools.partial(
        pl.kernel,
        mesh=mesh,
        out_type=jax.ShapeDtypeStruct((Btot, Dp), _F32),
        scratch_types=[
            pltpu.VMEM((CH,), jnp.int32),
            pltpu.VMEM((CH, Dp), _F32),
            pltpu.SemaphoreType.DMA,
        ],
    )
    def k(table_hbm, idx_hbm, out_hbm, idx_v, rows_v, sem):
        wid = lax.axis_index("s") * info.num_cores + lax.axis_index("c")

        def chunk(i, carry):
            base = wid * b_per_w + i * CH
            pltpu.sync_copy(idx_hbm.at[pl.ds(base, CH)], idx_v)
            pltpu.async_copy(table_hbm.at[idx_v], rows_v, sem).wait()
            pltpu.sync_copy(rows_v, out_hbm.at[pl.ds(base, CH)])
            return carry

        lax.fori_loop(0, nch, chunk, 0)

    return k(table, idx)


def _sa_dense(gath, centers_pad, WfT, bf, WaT, ba, C, Mt):
    B, M, K_, Dp = gath.shape
    Cf = WfT.shape[1]
    Cin = 3 + C

    def body(g_ref, c_ref, wf_ref, bf_ref, wa_ref, ba_ref, o_ref):
        g = g_ref[0]
        cen = c_ref[0][:, :3]
        local = g[:, :, :3] - cen[:, None, :]
        cat = jnp.concatenate([local, g[:, :, 3:3 + C]], axis=2)
        x2 = cat.reshape(Mt * K_, Cin)
        f = jnp.maximum(
            jnp.dot(x2, wf_ref[...], preferred_element_type=_F32) + bf_ref[...], 0.0)
        f3 = f.reshape(Mt, K_, Cf)
        fm = jnp.mean(f3, axis=1, keepdims=True)
        ax = jnp.concatenate([cat, f3 - fm], axis=2).reshape(Mt * K_, Cin + Cf)
        a = jnp.maximum(
            jnp.dot(ax, wa_ref[...], preferred_element_type=_F32) + ba_ref[...], 0.0)
        a3 = a.reshape(Mt, K_, Cf)
        amax = jnp.max(a3, axis=1, keepdims=True)
        e = jnp.exp(a3 - amax)
        w = e / jnp.sum(e, axis=1, keepdims=True)
        o_ref[0] = jnp.sum(w * f3, axis=1)

    return pl.pallas_call(
        body,
        grid=(B, M // Mt),
        in_specs=[
            pl.BlockSpec((1, Mt, K_, Dp), lambda b, m: (b, m, 0, 0)),
            pl.BlockSpec((1, Mt, 8), lambda b, m: (b, m, 0)),
            pl.BlockSpec(WfT.shape, lambda b, m: (0, 0)),
            pl.BlockSpec(bf.shape, lambda b, m: (0, 0)),
            pl.BlockSpec(WaT.shape, lambda b, m: (0, 0)),
            pl.BlockSpec(ba.shape, lambda b, m: (0, 0)),
        ],
        out_specs=pl.BlockSpec((1, Mt, Cf), lambda b, m: (b, m, 0)),
        out_shape=jax.ShapeDtypeStruct((B, M, Cf), _F32),
    )(gath, centers_pad, WfT, bf, WaT, ba)


def _fp(xyzlT_pad, xyzh_pad, featlT, fhT, W, b, Mt, head=None):
    B = xyzlT_pad.shape[0]
    Nl = xyzlT_pad.shape[2]
    Nh = xyzh_pad.shape[1]
    Ch = fhT.shape[1]
    Cl = featlT.shape[1]
    Cout = W.shape[0]
    hw = head if head is not None else ()
    n_out = hw[2].shape[0] if head is not None else Cout

    def body(*refs):
        cT_ref, ph_ref, flT_ref, fhT_ref, w_ref, b_ref = refs[:6]
        o_ref = refs[-1]
        cT = cT_ref[0]
        p = ph_ref[0]
        cc = jnp.sum(cT * cT, axis=0, keepdims=True)
        pp = jnp.sum(p * p, axis=1, keepdims=True)
        d = jnp.sqrt(jnp.maximum(
            pp + cc - 2.0 * jnp.dot(p, cT, preferred_element_type=_F32), 0.0))
        sub = lax.broadcasted_iota(jnp.int32, (Nh, Mt), 0)
        sels, ws = [], []
        for _ in range(3):
            mv = jnp.min(d, axis=0, keepdims=True)
            sel = jnp.min(jnp.where(d <= mv, sub, Nh), axis=0, keepdims=True)
            ws.append(1.0 / jnp.maximum(mv, 1e-8))
            sels.append(sel)
            d = jnp.where(sub == sel, jnp.float32(jnp.inf), d)
        wsum = ws[0] + ws[1] + ws[2]
        WiT = ((ws[0] / wsum) * (sub == sels[0]).astype(_F32)
               + (ws[1] / wsum) * (sub == sels[1]).astype(_F32)
               + (ws[2] / wsum) * (sub == sels[2]).astype(_F32))
        fiT = jnp.dot(fhT_ref[0], WiT, preferred_element_type=_F32)
        x = jnp.concatenate([fiT, flT_ref[0]], axis=0)
        u = jnp.maximum(
            jnp.dot(w_ref[...], x, preferred_element_type=_F32) + b_ref[...], 0.0)
        if head is not None:
            h1w_ref, h1b_ref, h2w_ref, h2b_ref = refs[6:10]
            h = jnp.maximum(
                jnp.dot(h1w_ref[...], u, preferred_element_type=_F32)
                + h1b_ref[...], 0.0)
            u = (jnp.dot(h2w_ref[...], h, preferred_element_type=_F32)
                 + h2b_ref[...])
        o_ref[0] = u

    in_specs = [
        pl.BlockSpec((1, 8, Mt), lambda bb, m: (bb, 0, m)),
        pl.BlockSpec((1, Nh, 8), lambda bb, m: (bb, 0, 0)),
        pl.BlockSpec((1, Cl, Mt), lambda bb, m: (bb, 0, m)),
        pl.BlockSpec((1, Ch, Nh), lambda bb, m: (bb, 0, 0)),
        pl.BlockSpec(W.shape, lambda bb, m: (0, 0)),
        pl.BlockSpec(b.shape, lambda bb, m: (0, 0)),
    ]
    args = [xyzlT_pad, xyzh_pad, featlT, fhT, W, b]
    for w_ in hw:
        in_specs.append(pl.BlockSpec(w_.shape, lambda bb, m: (0, 0)))
        args.append(w_)

    return pl.pallas_call(
        body,
        grid=(B, Nl // Mt),
        in_specs=in_specs,
        out_specs=pl.BlockSpec((1, n_out, Mt), lambda bb, m: (bb, 0, m)),
        out_shape=jax.ShapeDtypeStruct((B, n_out, Nl), _F32),
    )(*args)


def _centers_idx(N, M):
    import numpy as np
    return jnp.asarray(np.linspace(0, N - 1, M).astype(np.int32))


def _sa_layer(xyz_pad, table, pf, pa, M, Mt_knn, Mt_dense):
    B, N, Dp = table.shape
    idxc = _centers_idx(N, M)
    centers_pad = xyz_pad[:, idxc, :]
    centersT_pad = jnp.transpose(centers_pad, (0, 2, 1))
    idx = _knn(centersT_pad, xyz_pad, _K, Mt_knn)
    idx = jnp.transpose(idx, (0, 2, 1))
    gath = _sc_gather(table.reshape(B * N, Dp), idx.reshape(-1))
    gath = gath.reshape(B, M, _K, Dp)
    WfT, bf = _fold(pf)
    WaT, ba = _fold(pa)
    Cfeat = WfT.shape[0] - 3
    f = _sa_dense(gath, centers_pad, WfT, bf, WaT, ba, Cfeat, Mt_dense)
    return centers_pad, f


def kernel(X, params):
    B, N, _ = X.shape
    xyz = X[:, :, :3]
    xyz_pad = _pad_cols(xyz, 8)

    WsT, bs = _fold(params['spfe'])
    W1, W2 = WsT[0:3], WsT[3:6]
    Wc = _pad_cols((W1 + W2).T, 8).T
    Wm = _pad_cols(W2.T, 8).T
    feat0 = _spfe(xyz_pad, Wc, Wm, bs)

    M1, M2, M3 = N // 4, N // 8, N // 16

    T1 = _pad_cols(jnp.concatenate([xyz, feat0], axis=-1), 128)
    c1_pad, f1 = _sa_layer(xyz_pad, T1, params['sa1_f'], params['sa1_a'],
                           M1, 256, 128)
    xyz1 = c1_pad[:, :, :3]

    T2 = _pad_cols(jnp.concatenate([xyz1, f1, xyz1], axis=-1), 256)
    c2_pad, f2 = _sa_layer(c1_pad, T2, params['sa2_f'], params['sa2_a'],
                           M2, 256, 128)
    xyz2 = c2_pad[:, :, :3]

    T3 = _pad_cols(jnp.concatenate([xyz2, f2, xyz2], axis=-1), 384)
    c3_pad, f3 = _sa_layer(c2_pad, T3, params['sa3_f'], params['sa3_a'],
                           M3, 256, 128)

    def fold_cf(p):
        s = p['g'] / jnp.sqrt(1.0 + _EPS)
        return p['W'] * s[:, None], (p['b'] * s + p['bb'])[:, None]

    c1T = jnp.transpose(c1_pad, (0, 2, 1))
    c2T = jnp.transpose(c2_pad, (0, 2, 1))
    f1T = jnp.transpose(f1, (0, 2, 1))
    f2T = jnp.transpose(f2, (0, 2, 1))
    f3T = jnp.transpose(f3, (0, 2, 1))
    feat0T = jnp.transpose(feat0, (0, 2, 1))
    xyzT_pad = jnp.transpose(xyz_pad, (0, 2, 1))

    W3, b3 = fold_cf(params['fp3'])
    u3 = _fp(c2T, c3_pad, f2T, f3T, W3, b3, 256)
    W2, b2 = fold_cf(params['fp2'])
    u2 = _fp(c1T, c2_pad, f1T, u3, W2, b2, 256)
    W1, b1 = fold_cf(params['fp1'])
    H1, h1b = fold_cf(params['head1'])
    H2 = params['head2']['W']
    h2b = params['head2']['b'][:, None]
    return _fp(xyzT_pad, c1_pad, feat0T, u2, W1, b1, 256,
               head=(H1, h1b, H2, h2b))


# speedup vs baseline: 1.0101x; 1.0101x over previous
import functools

import jax
import jax.numpy as jnp
from jax import lax
from jax.experimental import pallas as pl
from jax.experimental.pallas import tpu as pltpu
from jax.experimental.pallas import tpu_sc as plsc

_EPS = 1e-5
_K = 32
_F32 = jnp.float32


def _fold(p):
    s = p['g'] / jnp.sqrt(1.0 + _EPS)
    W = p['W'] * s[:, None]
    b = p['b'] * s + p['bb']
    return W.T, b[None, :]


def _pad_cols(x, to):
    c = x.shape[-1]
    if c == to:
        return x
    pad = [(0, 0)] * (x.ndim - 1) + [(0, to - c)]
    return jnp.pad(x, pad)


def _spfe(xyz_pad, Wc, Wm, b):
    B, N, _ = xyz_pad.shape
    Cout = Wc.shape[1]

    def body(x_ref, wc_ref, wm_ref, b_ref, o_ref):
        x = x_ref[0]
        m = jnp.mean(x, axis=0, keepdims=True)
        y = (jnp.dot(x, wc_ref[...], preferred_element_type=_F32)
             - jnp.dot(m, wm_ref[...], preferred_element_type=_F32)
             + b_ref[...])
        o_ref[0] = jnp.maximum(y, 0.0)

    return pl.pallas_call(
        body,
        grid=(B,),
        in_specs=[
            pl.BlockSpec((1, N, 8), lambda i: (i, 0, 0)),
            pl.BlockSpec((8, Cout), lambda i: (0, 0)),
            pl.BlockSpec((8, Cout), lambda i: (0, 0)),
            pl.BlockSpec((1, Cout), lambda i: (0, 0)),
        ],
        out_specs=pl.BlockSpec((1, N, Cout), lambda i: (i, 0, 0)),
        out_shape=jax.ShapeDtypeStruct((B, N, Cout), _F32),
    )(xyz_pad, Wc, Wm, b)


def _knn(centersT_pad, xyz_pad, k, Mt):
    B = centersT_pad.shape[0]
    M = centersT_pad.shape[2]
    N = xyz_pad.shape[1]

    cw = 128
    nc = N // cw
    T = 8 if nc >= 32 else 16
    chunked = nc >= 8
    Csz = nc * T
    INF = float('inf')

    def body(p_ref, c_ref, idx_ref, d_ref, cv_ref, ci_ref):
        b = pl.program_id(0)
        p = p_ref[0]
        cT = c_ref[0]
        pp = jnp.sum(p * p, axis=1, keepdims=True)
        cc = jnp.sum(cT * cT, axis=0, keepdims=True)
        d2 = pp + cc - 2.0 * jnp.dot(p, cT, preferred_element_type=_F32)
        d_ref[...] = jnp.sqrt(jnp.maximum(d2, 0.0))
        sub = lax.broadcasted_iota(jnp.int32, (N, Mt), 0)
        krow = lax.broadcasted_iota(jnp.int32, (k, Mt), 0)

        def full_extract():
            def it(i, acc):
                d_ = d_ref[...]
                mv = jnp.min(d_, axis=0, keepdims=True)
                sel = jnp.min(jnp.where(d_ <= mv, sub, N), axis=0,
                              keepdims=True)
                acc = jnp.where(krow == i, sel, acc)
                d_ref[...] = jnp.where(sub == sel, INF, d_)
                return acc

            return lax.fori_loop(0, k, it, jnp.zeros((k, Mt), jnp.int32))

        if not chunked:
            idx_ref[0] = full_extract() + b * N
            return

        subw = lax.broadcasted_iota(jnp.int32, (cw, Mt), 0)
        trow = lax.broadcasted_iota(jnp.int32, (T, Mt), 0)

        def per_chunk(c, carry):
            slab = d_ref[pl.ds(c * cw, cw), :]
            cand_v = jnp.full((T, Mt), INF, _F32)
            cand_i = jnp.zeros((T, Mt), jnp.int32)
            for t in range(T):
                mv = jnp.min(slab, axis=0, keepdims=True)
                sel = jnp.min(jnp.where(slab <= mv, subw, cw), axis=0,
                              keepdims=True)
                cand_v = jnp.where(trow == t, mv, cand_v)
                cand_i = jnp.where(trow == t, sel + c * cw, cand_i)
                slab = jnp.where(subw == sel, INF, slab)
            cv_ref[pl.ds(c * T, T), :] = cand_v
            ci_ref[pl.ds(c * T, T), :] = cand_i
            return carry

        lax.fori_loop(0, nc, per_chunk, 0)

        sub5 = lax.broadcasted_iota(jnp.int32, (Csz, Mt), 0)

        def rnd(i, carry):
            acc, bad = carry
            cv = cv_ref[...]
            mv = jnp.min(cv, axis=0, keepdims=True)
            sel = jnp.min(jnp.where(cv <= mv, sub5, Csz), axis=0,
                          keepdims=True)
            hit = sub5 == sel
            idxsel = jnp.min(jnp.where(hit, ci_ref[...],
                                       jnp.int32(0x7FFFFFFF)),
                             axis=0, keepdims=True)
            acc = jnp.where(krow == i, idxsel, acc)
            bad = jnp.maximum(bad, (sel % T == T - 1).astype(jnp.int32))
            cv_ref[...] = jnp.where(hit, INF, cv)
            return acc, bad

        acc, bad = lax.fori_loop(
            0, k, rnd,
            (jnp.zeros((k, Mt), jnp.int32), jnp.zeros((1, Mt), jnp.int32)))
        idx_ref[0] = acc + b * N

        @pl.when(jnp.max(bad) > 0)
        def _():
            idx_ref[0] = full_extract() + b * N

    return pl.pallas_call(
        body,
        grid=(B, M // Mt),
        in_specs=[
            pl.BlockSpec((1, N, 8), lambda b, m: (b, 0, 0)),
            pl.BlockSpec((1, 8, Mt), lambda b, m: (b, 0, m)),
        ],
        out_specs=pl.BlockSpec((1, k, Mt), lambda b, m: (b, 0, m)),
        out_shape=jax.ShapeDtypeStruct((B, k, M), jnp.int32),
        scratch_shapes=[
            pltpu.VMEM((N, Mt), _F32),
            pltpu.VMEM((max(Csz, 8), Mt), _F32),
            pltpu.VMEM((max(Csz, 8), Mt), jnp.int32),
        ],
    )(xyz_pad, centersT_pad)


def _sc_gather(table, idx):
    R, Dp = table.shape
    (Btot,) = idx.shape
    info = plsc.get_sparse_core_info()
    NW = info.num_cores * info.num_subcores
    CH = 128
    b_per_w = Btot // NW
    nch = b_per_w // CH
    mesh = plsc.VectorSubcoreMesh(core_axis_name="c", subcore_axis_name="s")

    nbuf = 2

    @functools.partial(
        pl.kernel,
        mesh=mesh,
        out_type=jax.ShapeDtypeStruct((Btot, Dp), _F32),
        scratch_types=[
            pltpu.VMEM((b_per_w,), jnp.int32),
            pltpu.VMEM((nbuf, CH, Dp), _F32),
            pltpu.SemaphoreType.DMA,
            pltpu.SemaphoreType.DMA,
        ],
    )
    def k(table_hbm, idx_hbm, out_hbm, idx_all, rows_v, sem0, sem1):
        wid = lax.axis_index("s") * info.num_cores + lax.axis_index("c")
        wbase = wid * b_per_w
        sems = (sem0, sem1)
        pltpu.sync_copy(idx_hbm.at[pl.ds(wbase, b_per_w)], idx_all)
        for b in range(nbuf):
            pltpu.async_copy(
                table_hbm.at[idx_all.at[pl.ds(b * CH, CH)]],
                rows_v.at[b], sems[b])

        def round_(r, carry):
            for b in range(nbuf):
                i = r * nbuf + b
                pltpu.make_async_copy(
                    table_hbm.at[idx_all.at[pl.ds(i * CH, CH)]],
                    rows_v.at[b], sems[b]).wait()
                pltpu.sync_copy(rows_v.at[b],
                                out_hbm.at[pl.ds(wbase + i * CH, CH)])

                @pl.when(i + nbuf < nch)
                def _():
                    pltpu.async_copy(
                        table_hbm.at[idx_all.at[pl.ds((i + nbuf) * CH, CH)]],
                        rows_v.at[b], sems[b])
            return carry

        lax.fori_loop(0, nch // nbuf, round_, 0)

    return k(table, idx)


def _sa_dense(gath, centers_pad, WfT, bf, WaT, ba, C, Mt):
    B, M, K_, Dp = gath.shape
    Cf = WfT.shape[1]
    Cin = 3 + C

    def body(g_ref, c_ref, wf_ref, bf_ref, wa_ref, ba_ref, o_ref):
        g = g_ref[0]
        cen = c_ref[0][:, :3]
        local = g[:, :, :3] - cen[:, None, :]
        cat = jnp.concatenate([local, g[:, :, 3:3 + C]], axis=2)
        x2 = cat.reshape(Mt * K_, Cin)
        f = jnp.maximum(
            jnp.dot(x2, wf_ref[...], preferred_element_type=_F32) + bf_ref[...], 0.0)
        f3 = f.reshape(Mt, K_, Cf)
        fm = jnp.mean(f3, axis=1, keepdims=True)
        ax = jnp.concatenate([cat, f3 - fm], axis=2).reshape(Mt * K_, Cin + Cf)
        a = jnp.maximum(
            jnp.dot(ax, wa_ref[...], preferred_element_type=_F32) + ba_ref[...], 0.0)
        a3 = a.reshape(Mt, K_, Cf)
        amax = jnp.max(a3, axis=1, keepdims=True)
        e = jnp.exp(a3 - amax)
        w = e / jnp.sum(e, axis=1, keepdims=True)
        o_ref[0] = jnp.sum(w * f3, axis=1)

    return pl.pallas_call(
        body,
        grid=(B, M // Mt),
        in_specs=[
            pl.BlockSpec((1, Mt, K_, Dp), lambda b, m: (b, m, 0, 0)),
            pl.BlockSpec((1, Mt, 8), lambda b, m: (b, m, 0)),
            pl.BlockSpec(WfT.shape, lambda b, m: (0, 0)),
            pl.BlockSpec(bf.shape, lambda b, m: (0, 0)),
            pl.BlockSpec(WaT.shape, lambda b, m: (0, 0)),
            pl.BlockSpec(ba.shape, lambda b, m: (0, 0)),
        ],
        out_specs=pl.BlockSpec((1, Mt, Cf), lambda b, m: (b, m, 0)),
        out_shape=jax.ShapeDtypeStruct((B, M, Cf), _F32),
    )(gath, centers_pad, WfT, bf, WaT, ba)


def _fp(xyzlT_pad, xyzh_pad, featlT, fhT, W, b, Mt, head=None):
    B = xyzlT_pad.shape[0]
    Nl = xyzlT_pad.shape[2]
    Nh = xyzh_pad.shape[1]
    Ch = fhT.shape[1]
    Cl = featlT.shape[1]
    Cout = W.shape[0]
    hw = head if head is not None else ()
    n_out = hw[2].shape[0] if head is not None else Cout

    def body(*refs):
        cT_ref, ph_ref, flT_ref, fhT_ref, w_ref, b_ref = refs[:6]
        o_ref = refs[-1]
        cT = cT_ref[0]
        p = ph_ref[0]
        cc = jnp.sum(cT * cT, axis=0, keepdims=True)
        pp = jnp.sum(p * p, axis=1, keepdims=True)
        d = jnp.sqrt(jnp.maximum(
            pp + cc - 2.0 * jnp.dot(p, cT, preferred_element_type=_F32), 0.0))
        sub = lax.broadcasted_iota(jnp.int32, (Nh, Mt), 0)
        sels, ws = [], []
        for _ in range(3):
            mv = jnp.min(d, axis=0, keepdims=True)
            sel = jnp.min(jnp.where(d <= mv, sub, Nh), axis=0, keepdims=True)
            ws.append(1.0 / jnp.maximum(mv, 1e-8))
            sels.append(sel)
            d = jnp.where(sub == sel, jnp.float32(jnp.inf), d)
        wsum = ws[0] + ws[1] + ws[2]
        WiT = ((ws[0] / wsum) * (sub == sels[0]).astype(_F32)
               + (ws[1] / wsum) * (sub == sels[1]).astype(_F32)
               + (ws[2] / wsum) * (sub == sels[2]).astype(_F32))
        fiT = jnp.dot(fhT_ref[0], WiT, preferred_element_type=_F32)
        x = jnp.concatenate([fiT, flT_ref[0]], axis=0)
        u = jnp.maximum(
            jnp.dot(w_ref[...], x, preferred_element_type=_F32) + b_ref[...], 0.0)
        if head is not None:
            h1w_ref, h1b_ref, h2w_ref, h2b_ref = refs[6:10]
            h = jnp.maximum(
                jnp.dot(h1w_ref[...], u, preferred_element_type=_F32)
                + h1b_ref[...], 0.0)
            u = (jnp.dot(h2w_ref[...], h, preferred_element_type=_F32)
                 + h2b_ref[...])
        o_ref[0] = u

    in_specs = [
        pl.BlockSpec((1, 8, Mt), lambda bb, m: (bb, 0, m)),
        pl.BlockSpec((1, Nh, 8), lambda bb, m: (bb, 0, 0)),
        pl.BlockSpec((1, Cl, Mt), lambda bb, m: (bb, 0, m)),
        pl.BlockSpec((1, Ch, Nh), lambda bb, m: (bb, 0, 0)),
        pl.BlockSpec(W.shape, lambda bb, m: (0, 0)),
        pl.BlockSpec(b.shape, lambda bb, m: (0, 0)),
    ]
    args = [xyzlT_pad, xyzh_pad, featlT, fhT, W, b]
    for w_ in hw:
        in_specs.append(pl.BlockSpec(w_.shape, lambda bb, m: (0, 0)))
        args.append(w_)

    return pl.pallas_call(
        body,
        grid=(B, Nl // Mt),
        in_specs=in_specs,
        out_specs=pl.BlockSpec((1, n_out, Mt), lambda bb, m: (bb, 0, m)),
        out_shape=jax.ShapeDtypeStruct((B, n_out, Nl), _F32),
    )(*args)


def _centers_idx(N, M):
    import numpy as np
    return jnp.asarray(np.linspace(0, N - 1, M).astype(np.int32))


def _sa_layer(xyz_pad, table, pf, pa, M, Mt_knn, Mt_dense):
    B, N, Dp = table.shape
    idxc = _centers_idx(N, M)
    centers_pad = xyz_pad[:, idxc, :]
    centersT_pad = jnp.transpose(centers_pad, (0, 2, 1))
    idx = _knn(centersT_pad, xyz_pad, _K, Mt_knn)
    idx = jnp.transpose(idx, (0, 2, 1))
    gath = _sc_gather(table.reshape(B * N, Dp), idx.reshape(-1))
    gath = gath.reshape(B, M, _K, Dp)
    WfT, bf = _fold(pf)
    WaT, ba = _fold(pa)
    Cfeat = WfT.shape[0] - 3
    f = _sa_dense(gath, centers_pad, WfT, bf, WaT, ba, Cfeat, Mt_dense)
    return centers_pad, f


def kernel(X, params):
    B, N, _ = X.shape
    xyz = X[:, :, :3]
    xyz_pad = _pad_cols(xyz, 8)

    WsT, bs = _fold(params['spfe'])
    W1, W2 = WsT[0:3], WsT[3:6]
    Wc = _pad_cols((W1 + W2).T, 8).T
    Wm = _pad_cols(W2.T, 8).T
    feat0 = _spfe(xyz_pad, Wc, Wm, bs)

    M1, M2, M3 = N // 4, N // 8, N // 16

    T1 = _pad_cols(jnp.concatenate([xyz, feat0], axis=-1), 128)
    c1_pad, f1 = _sa_layer(xyz_pad, T1, params['sa1_f'], params['sa1_a'],
                           M1, 256, 128)
    xyz1 = c1_pad[:, :, :3]

    T2 = _pad_cols(jnp.concatenate([xyz1, f1, xyz1], axis=-1), 256)
    c2_pad, f2 = _sa_layer(c1_pad, T2, params['sa2_f'], params['sa2_a'],
                           M2, 256, 128)
    xyz2 = c2_pad[:, :, :3]

    T3 = _pad_cols(jnp.concatenate([xyz2, f2, xyz2], axis=-1), 384)
    c3_pad, f3 = _sa_layer(c2_pad, T3, params['sa3_f'], params['sa3_a'],
                           M3, 256, 128)

    def fold_cf(p):
        s = p['g'] / jnp.sqrt(1.0 + _EPS)
        return p['W'] * s[:, None], (p['b'] * s + p['bb'])[:, None]

    c1T = jnp.transpose(c1_pad, (0, 2, 1))
    c2T = jnp.transpose(c2_pad, (0, 2, 1))
    f1T = jnp.transpose(f1, (0, 2, 1))
    f2T = jnp.transpose(f2, (0, 2, 1))
    f3T = jnp.transpose(f3, (0, 2, 1))
    feat0T = jnp.transpose(feat0, (0, 2, 1))
    xyzT_pad = jnp.transpose(xyz_pad, (0, 2, 1))

    W3, b3 = fold_cf(params['fp3'])
    u3 = _fp(c2T, c3_pad, f2T, f3T, W3, b3, 256)
    W2, b2 = fold_cf(params['fp2'])
    u2 = _fp(c1T, c2_pad, f1T, u3, W2, b2, 256)
    W1, b1 = fold_cf(params['fp1'])
    H1, h1b = fold_cf(params['head1'])
    H2 = params['head2']['W']
    h2b = params['head2']['b'][:, None]
    return _fp(xyzT_pad, c1_pad, feat0T, u2, W1, b1, 256,
               head=(H1, h1b, H2, h2b))


# speedup vs baseline: 1.0305x; 1.0202x over previous
import functools

import jax
import jax.numpy as jnp
from jax import lax
from jax.experimental import pallas as pl
from jax.experimental.pallas import tpu as pltpu
from jax.experimental.pallas import tpu_sc as plsc

_EPS = 1e-5
_K = 32
_F32 = jnp.float32


def _fold(p):
    s = p['g'] / jnp.sqrt(1.0 + _EPS)
    W = p['W'] * s[:, None]
    b = p['b'] * s + p['bb']
    return W.T, b[None, :]


def _pad_cols(x, to):
    c = x.shape[-1]
    if c == to:
        return x
    pad = [(0, 0)] * (x.ndim - 1) + [(0, to - c)]
    return jnp.pad(x, pad)


def _spfe(xyz_pad, Wc, Wm, b):
    B, N, _ = xyz_pad.shape
    Cout = Wc.shape[1]

    def body(x_ref, wc_ref, wm_ref, b_ref, o_ref):
        x = x_ref[0]
        m = jnp.mean(x, axis=0, keepdims=True)
        y = (jnp.dot(x, wc_ref[...], preferred_element_type=_F32)
             - jnp.dot(m, wm_ref[...], preferred_element_type=_F32)
             + b_ref[...])
        o_ref[0] = jnp.maximum(y, 0.0)

    return pl.pallas_call(
        body,
        grid=(B,),
        in_specs=[
            pl.BlockSpec((1, N, 8), lambda i: (i, 0, 0)),
            pl.BlockSpec((8, Cout), lambda i: (0, 0)),
            pl.BlockSpec((8, Cout), lambda i: (0, 0)),
            pl.BlockSpec((1, Cout), lambda i: (0, 0)),
        ],
        out_specs=pl.BlockSpec((1, N, Cout), lambda i: (i, 0, 0)),
        out_shape=jax.ShapeDtypeStruct((B, N, Cout), _F32),
    )(xyz_pad, Wc, Wm, b)


def _knn(centersT_pad, xyz_pad, k, Mt):
    B = centersT_pad.shape[0]
    M = centersT_pad.shape[2]
    N = xyz_pad.shape[1]

    cw = 128
    nc = N // cw
    T = 8 if nc >= 32 else 16
    chunked = nc >= 8
    Csz = nc * T
    INF = float('inf')

    def body(p_ref, c_ref, idx_ref, d_ref, cv_ref, ci_ref):
        b = pl.program_id(0)
        p = p_ref[0]
        cT = c_ref[0]
        pp = jnp.sum(p * p, axis=1, keepdims=True)
        cc = jnp.sum(cT * cT, axis=0, keepdims=True)
        d2 = pp + cc - 2.0 * jnp.dot(p, cT, preferred_element_type=_F32)
        d_ref[...] = jnp.sqrt(jnp.maximum(d2, 0.0))
        sub = lax.broadcasted_iota(jnp.int32, (N, Mt), 0)
        krow = lax.broadcasted_iota(jnp.int32, (k, Mt), 0)

        def full_extract():
            def it(i, acc):
                d_ = d_ref[...]
                mv = jnp.min(d_, axis=0, keepdims=True)
                sel = jnp.min(jnp.where(d_ <= mv, sub, N), axis=0,
                              keepdims=True)
                acc = jnp.where(krow == i, sel, acc)
                d_ref[...] = jnp.where(sub == sel, INF, d_)
                return acc

            return lax.fori_loop(0, k, it, jnp.zeros((k, Mt), jnp.int32))

        if not chunked:
            idx_ref[0] = full_extract() + b * N
            return

        subw = lax.broadcasted_iota(jnp.int32, (cw, Mt), 0)
        trow = lax.broadcasted_iota(jnp.int32, (T, Mt), 0)

        def per_chunk(c, carry):
            slab = d_ref[pl.ds(c * cw, cw), :]
            cand_v = jnp.full((T, Mt), INF, _F32)
            cand_i = jnp.zeros((T, Mt), jnp.int32)
            for t in range(T):
                mv = jnp.min(slab, axis=0, keepdims=True)
                sel = jnp.min(jnp.where(slab <= mv, subw, cw), axis=0,
                              keepdims=True)
                cand_v = jnp.where(trow == t, mv, cand_v)
                cand_i = jnp.where(trow == t, sel + c * cw, cand_i)
                slab = jnp.where(subw == sel, INF, slab)
            cv_ref[pl.ds(c * T, T), :] = cand_v
            ci_ref[pl.ds(c * T, T), :] = cand_i
            return carry

        lax.fori_loop(0, nc, per_chunk, 0)

        sub5 = lax.broadcasted_iota(jnp.int32, (Csz, Mt), 0)

        def rnd(i, carry):
            acc, bad = carry
            cv = cv_ref[...]
            mv = jnp.min(cv, axis=0, keepdims=True)
            sel = jnp.min(jnp.where(cv <= mv, sub5, Csz), axis=0,
                          keepdims=True)
            hit = sub5 == sel
            idxsel = jnp.min(jnp.where(hit, ci_ref[...],
                                       jnp.int32(0x7FFFFFFF)),
                             axis=0, keepdims=True)
            acc = jnp.where(krow == i, idxsel, acc)
            bad = jnp.maximum(bad, (sel % T == T - 1).astype(jnp.int32))
            cv_ref[...] = jnp.where(hit, INF, cv)
            return acc, bad

        acc, bad = lax.fori_loop(
            0, k, rnd,
            (jnp.zeros((k, Mt), jnp.int32), jnp.zeros((1, Mt), jnp.int32)))
        idx_ref[0] = acc + b * N

        @pl.when(jnp.max(bad) > 0)
        def _():
            idx_ref[0] = full_extract() + b * N

    return pl.pallas_call(
        body,
        grid=(B, M // Mt),
        in_specs=[
            pl.BlockSpec((1, N, 8), lambda b, m: (b, 0, 0)),
            pl.BlockSpec((1, 8, Mt), lambda b, m: (b, 0, m)),
        ],
        out_specs=pl.BlockSpec((1, k, Mt), lambda b, m: (b, 0, m)),
        out_shape=jax.ShapeDtypeStruct((B, k, M), jnp.int32),
        scratch_shapes=[
            pltpu.VMEM((N, Mt), _F32),
            pltpu.VMEM((max(Csz, 8), Mt), _F32),
            pltpu.VMEM((max(Csz, 8), Mt), jnp.int32),
        ],
    )(xyz_pad, centersT_pad)


def _sc_gather(table, idx):
    R, Dp = table.shape
    (Btot,) = idx.shape
    info = plsc.get_sparse_core_info()
    NW = info.num_cores * info.num_subcores
    CH = 128
    b_per_w = Btot // NW
    nch = b_per_w // CH
    mesh = plsc.VectorSubcoreMesh(core_axis_name="c", subcore_axis_name="s")

    nbuf = 2

    @functools.partial(
        pl.kernel,
        mesh=mesh,
        out_type=jax.ShapeDtypeStruct((Btot, Dp), _F32),
        scratch_types=[
            pltpu.VMEM((b_per_w,), jnp.int32),
            pltpu.VMEM((nbuf, CH, Dp), _F32),
            pltpu.SemaphoreType.DMA,
            pltpu.SemaphoreType.DMA,
        ],
    )
    def k(table_hbm, idx_hbm, out_hbm, idx_all, rows_v, sem0, sem1):
        wid = lax.axis_index("s") * info.num_cores + lax.axis_index("c")
        wbase = wid * b_per_w
        sems = (sem0, sem1)
        pltpu.sync_copy(idx_hbm.at[pl.ds(wbase, b_per_w)], idx_all)
        for b in range(nbuf):
            pltpu.async_copy(
                table_hbm.at[idx_all.at[pl.ds(b * CH, CH)]],
                rows_v.at[b], sems[b])

        def round_(r, carry):
            for b in range(nbuf):
                i = r * nbuf + b
                pltpu.make_async_copy(
                    table_hbm.at[idx_all.at[pl.ds(i * CH, CH)]],
                    rows_v.at[b], sems[b]).wait()
                pltpu.sync_copy(rows_v.at[b],
                                out_hbm.at[pl.ds(wbase + i * CH, CH)])

                @pl.when(i + nbuf < nch)
                def _():
                    pltpu.async_copy(
                        table_hbm.at[idx_all.at[pl.ds((i + nbuf) * CH, CH)]],
                        rows_v.at[b], sems[b])
            return carry

        lax.fori_loop(0, nch // nbuf, round_, 0)

    return k(table, idx)


def _sa_dense(gath, centers_pad, WfT, bf, WaT, ba, C, Mt):
    B, M, K_, Dp = gath.shape
    Cf = WfT.shape[1]
    Cin = 3 + C

    def body(g_ref, c_ref, wf_ref, bf_ref, wa_ref, ba_ref, o_ref):
        g = g_ref[0]
        cen = c_ref[0][:, :3]
        local = g[:, :, :3] - cen[:, None, :]
        cat = jnp.concatenate([local, g[:, :, 3:3 + C]], axis=2)
        x2 = cat.reshape(Mt * K_, Cin)
        f = jnp.maximum(
            jnp.dot(x2, wf_ref[...], preferred_element_type=_F32) + bf_ref[...], 0.0)
        f3 = f.reshape(Mt, K_, Cf)
        fm = jnp.mean(f3, axis=1, keepdims=True)
        ax = jnp.concatenate([cat, f3 - fm], axis=2).reshape(Mt * K_, Cin + Cf)
        a = jnp.maximum(
            jnp.dot(ax, wa_ref[...], preferred_element_type=_F32) + ba_ref[...], 0.0)
        a3 = a.reshape(Mt, K_, Cf)
        amax = jnp.max(a3, axis=1, keepdims=True)
        e = jnp.exp(a3 - amax)
        w = e / jnp.sum(e, axis=1, keepdims=True)
        o_ref[0] = jnp.sum(w * f3, axis=1)

    return pl.pallas_call(
        body,
        grid=(B, M // Mt),
        in_specs=[
            pl.BlockSpec((1, Mt, K_, Dp), lambda b, m: (b, m, 0, 0)),
            pl.BlockSpec((1, Mt, 8), lambda b, m: (b, m, 0)),
            pl.BlockSpec(WfT.shape, lambda b, m: (0, 0)),
            pl.BlockSpec(bf.shape, lambda b, m: (0, 0)),
            pl.BlockSpec(WaT.shape, lambda b, m: (0, 0)),
            pl.BlockSpec(ba.shape, lambda b, m: (0, 0)),
        ],
        out_specs=pl.BlockSpec((1, Mt, Cf), lambda b, m: (b, m, 0)),
        out_shape=jax.ShapeDtypeStruct((B, M, Cf), _F32),
    )(gath, centers_pad, WfT, bf, WaT, ba)


def _fp(xyzlT_pad, xyzh_pad, featlT, fhT, W, b, Mt, head=None):
    B = xyzlT_pad.shape[0]
    Nl = xyzlT_pad.shape[2]
    Nh = xyzh_pad.shape[1]
    Ch = fhT.shape[1]
    Cl = featlT.shape[1]
    Cout = W.shape[0]
    hw = head if head is not None else ()
    n_out = hw[2].shape[0] if head is not None else Cout

    def body(*refs):
        cT_ref, ph_ref, flT_ref, fhT_ref, w_ref, b_ref = refs[:6]
        o_ref = refs[-1]
        cT = cT_ref[0]
        p = ph_ref[0]
        cc = jnp.sum(cT * cT, axis=0, keepdims=True)
        pp = jnp.sum(p * p, axis=1, keepdims=True)
        d = jnp.sqrt(jnp.maximum(
            pp + cc - 2.0 * jnp.dot(p, cT, preferred_element_type=_F32), 0.0))
        sub = lax.broadcasted_iota(jnp.int32, (Nh, Mt), 0)
        sels, ws = [], []
        for _ in range(3):
            mv = jnp.min(d, axis=0, keepdims=True)
            sel = jnp.min(jnp.where(d <= mv, sub, Nh), axis=0, keepdims=True)
            ws.append(1.0 / jnp.maximum(mv, 1e-8))
            sels.append(sel)
            d = jnp.where(sub == sel, jnp.float32(jnp.inf), d)
        wsum = ws[0] + ws[1] + ws[2]
        WiT = ((ws[0] / wsum) * (sub == sels[0]).astype(_F32)
               + (ws[1] / wsum) * (sub == sels[1]).astype(_F32)
               + (ws[2] / wsum) * (sub == sels[2]).astype(_F32))
        fiT = jnp.dot(fhT_ref[0], WiT, preferred_element_type=_F32)
        x = jnp.concatenate([fiT, flT_ref[0]], axis=0)
        u = jnp.maximum(
            jnp.dot(w_ref[...], x, preferred_element_type=_F32) + b_ref[...], 0.0)
        if head is not None:
            h1w_ref, h1b_ref, h2w_ref, h2b_ref = refs[6:10]
            h = jnp.maximum(
                jnp.dot(h1w_ref[...], u, preferred_element_type=_F32)
                + h1b_ref[...], 0.0)
            u = (jnp.dot(h2w_ref[...], h, preferred_element_type=_F32)
                 + h2b_ref[...])
        o_ref[0] = u

    in_specs = [
        pl.BlockSpec((1, 8, Mt), lambda bb, m: (bb, 0, m)),
        pl.BlockSpec((1, Nh, 8), lambda bb, m: (bb, 0, 0)),
        pl.BlockSpec((1, Cl, Mt), lambda bb, m: (bb, 0, m)),
        pl.BlockSpec((1, Ch, Nh), lambda bb, m: (bb, 0, 0)),
        pl.BlockSpec(W.shape, lambda bb, m: (0, 0)),
        pl.BlockSpec(b.shape, lambda bb, m: (0, 0)),
    ]
    args = [xyzlT_pad, xyzh_pad, featlT, fhT, W, b]
    for w_ in hw:
        in_specs.append(pl.BlockSpec(w_.shape, lambda bb, m: (0, 0)))
        args.append(w_)

    return pl.pallas_call(
        body,
        grid=(B, Nl // Mt),
        in_specs=in_specs,
        out_specs=pl.BlockSpec((1, n_out, Mt), lambda bb, m: (bb, 0, m)),
        out_shape=jax.ShapeDtypeStruct((B, n_out, Nl), _F32),
    )(*args)


def _centers_idx(N, M):
    import numpy as np
    return jnp.asarray(np.linspace(0, N - 1, M).astype(np.int32))


def _sa_layer(xyz_pad, table, pf, pa, M, Mt_knn, Mt_dense):
    B, N, Dp = table.shape
    idxc = _centers_idx(N, M)
    centers_pad = xyz_pad[:, idxc, :]
    centersT_pad = jnp.transpose(centers_pad, (0, 2, 1))
    idx = _knn(centersT_pad, xyz_pad, _K, Mt_knn)
    idx = jnp.transpose(idx, (0, 2, 1))
    gath = _sc_gather(table.reshape(B * N, Dp), idx.reshape(-1))
    gath = gath.reshape(B, M, _K, Dp)
    WfT, bf = _fold(pf)
    WaT, ba = _fold(pa)
    Cfeat = WfT.shape[0] - 3
    f = _sa_dense(gath, centers_pad, WfT, bf, WaT, ba, Cfeat, Mt_dense)
    return centers_pad, f


def kernel(X, params):
    B, N, _ = X.shape
    xyz = X[:, :, :3]
    xyz_pad = _pad_cols(xyz, 8)

    WsT, bs = _fold(params['spfe'])
    W1, W2 = WsT[0:3], WsT[3:6]
    Wc = _pad_cols((W1 + W2).T, 8).T
    Wm = _pad_cols(W2.T, 8).T
    feat0 = _spfe(xyz_pad, Wc, Wm, bs)

    M1, M2, M3 = N // 4, N // 8, N // 16

    T1 = _pad_cols(jnp.concatenate([xyz, feat0], axis=-1), 128)
    c1_pad, f1 = _sa_layer(xyz_pad, T1, params['sa1_f'], params['sa1_a'],
                           M1, 512, 128)
    xyz1 = c1_pad[:, :, :3]

    T2 = _pad_cols(jnp.concatenate([xyz1, f1, xyz1], axis=-1), 256)
    c2_pad, f2 = _sa_layer(c1_pad, T2, params['sa2_f'], params['sa2_a'],
                           M2, 512, 128)
    xyz2 = c2_pad[:, :, :3]

    T3 = _pad_cols(jnp.concatenate([xyz2, f2, xyz2], axis=-1), 384)
    c3_pad, f3 = _sa_layer(c2_pad, T3, params['sa3_f'], params['sa3_a'],
                           M3, 512, 128)

    def fold_cf(p):
        s = p['g'] / jnp.sqrt(1.0 + _EPS)
        return p['W'] * s[:, None], (p['b'] * s + p['bb'])[:, None]

    c1T = jnp.transpose(c1_pad, (0, 2, 1))
    c2T = jnp.transpose(c2_pad, (0, 2, 1))
    f1T = jnp.transpose(f1, (0, 2, 1))
    f2T = jnp.transpose(f2, (0, 2, 1))
    f3T = jnp.transpose(f3, (0, 2, 1))
    feat0T = jnp.transpose(feat0, (0, 2, 1))
    xyzT_pad = jnp.transpose(xyz_pad, (0, 2, 1))

    W3, b3 = fold_cf(params['fp3'])
    u3 = _fp(c2T, c3_pad, f2T, f3T, W3, b3, 256)
    W2, b2 = fold_cf(params['fp2'])
    u2 = _fp(c1T, c2_pad, f1T, u3, W2, b2, 256)
    W1, b1 = fold_cf(params['fp1'])
    H1, h1b = fold_cf(params['head1'])
    H2 = params['head2']['W']
    h2b = params['head2']['b'][:, None]
    return _fp(xyzT_pad, c1_pad, feat0T, u2, W1, b1, 256,
               head=(H1, h1b, H2, h2b))


# speedup vs baseline: 1.0641x; 1.0326x over previous
import functools

import jax
import jax.numpy as jnp
from jax import lax
from jax.experimental import pallas as pl
from jax.experimental.pallas import tpu as pltpu
from jax.experimental.pallas import tpu_sc as plsc

_EPS = 1e-5
_K = 32
_F32 = jnp.float32


def _fold(p):
    s = p['g'] / jnp.sqrt(1.0 + _EPS)
    W = p['W'] * s[:, None]
    b = p['b'] * s + p['bb']
    return W.T, b[None, :]


def _pad_cols(x, to):
    c = x.shape[-1]
    if c == to:
        return x
    pad = [(0, 0)] * (x.ndim - 1) + [(0, to - c)]
    return jnp.pad(x, pad)


def _spfe(xyz_pad, Wc, Wm, b):
    B, N, _ = xyz_pad.shape
    Cout = Wc.shape[1]

    def body(x_ref, wc_ref, wm_ref, b_ref, o_ref):
        x = x_ref[0]
        m = jnp.mean(x, axis=0, keepdims=True)
        y = (jnp.dot(x, wc_ref[...], preferred_element_type=_F32)
             - jnp.dot(m, wm_ref[...], preferred_element_type=_F32)
             + b_ref[...])
        o_ref[0] = jnp.maximum(y, 0.0)

    return pl.pallas_call(
        body,
        grid=(B,),
        in_specs=[
            pl.BlockSpec((1, N, 8), lambda i: (i, 0, 0)),
            pl.BlockSpec((8, Cout), lambda i: (0, 0)),
            pl.BlockSpec((8, Cout), lambda i: (0, 0)),
            pl.BlockSpec((1, Cout), lambda i: (0, 0)),
        ],
        out_specs=pl.BlockSpec((1, N, Cout), lambda i: (i, 0, 0)),
        out_shape=jax.ShapeDtypeStruct((B, N, Cout), _F32),
    )(xyz_pad, Wc, Wm, b)


def _knn(centersT_pad, xyz_pad, k, Mt):
    B = centersT_pad.shape[0]
    M = centersT_pad.shape[2]
    N = xyz_pad.shape[1]

    cw = 128
    nc = N // cw
    T = 8 if nc >= 32 else 16
    chunked = nc >= 8
    Csz = nc * T
    INF = float('inf')

    def body(p_ref, c_ref, idx_ref, d_ref, cv_ref, ci_ref):
        b = pl.program_id(0)
        p = p_ref[0]
        cT = c_ref[0]
        pp = jnp.sum(p * p, axis=1, keepdims=True)
        cc = jnp.sum(cT * cT, axis=0, keepdims=True)
        d2 = pp + cc - 2.0 * jnp.dot(p, cT, preferred_element_type=_F32)
        d_ref[...] = jnp.sqrt(jnp.maximum(d2, 0.0))
        sub = lax.broadcasted_iota(jnp.int32, (N, Mt), 0)
        krow = lax.broadcasted_iota(jnp.int32, (k, Mt), 0)

        def full_extract():
            def it(i, acc):
                d_ = d_ref[...]
                mv = jnp.min(d_, axis=0, keepdims=True)
                sel = jnp.min(jnp.where(d_ <= mv, sub, N), axis=0,
                              keepdims=True)
                acc = jnp.where(krow == i, sel, acc)
                d_ref[...] = jnp.where(sub == sel, INF, d_)
                return acc

            return lax.fori_loop(0, k, it, jnp.zeros((k, Mt), jnp.int32))

        if not chunked:
            idx_ref[0] = full_extract() + b * N
            return

        subw = lax.broadcasted_iota(jnp.int32, (cw, Mt), 0)
        trow = lax.broadcasted_iota(jnp.int32, (T, Mt), 0)

        def per_chunk(c, carry):
            slab = d_ref[pl.ds(c * cw, cw), :]
            cand_v = jnp.full((T, Mt), INF, _F32)
            cand_i = jnp.zeros((T, Mt), jnp.int32)
            for t in range(T):
                mv = jnp.min(slab, axis=0, keepdims=True)
                sel = jnp.min(jnp.where(slab <= mv, subw, cw), axis=0,
                              keepdims=True)
                cand_v = jnp.where(trow == t, mv, cand_v)
                cand_i = jnp.where(trow == t, sel + c * cw, cand_i)
                slab = jnp.where(subw == sel, INF, slab)
            cv_ref[pl.ds(c * T, T), :] = cand_v
            ci_ref[pl.ds(c * T, T), :] = cand_i
            return carry

        lax.fori_loop(0, nc, per_chunk, 0)

        sub5 = lax.broadcasted_iota(jnp.int32, (Csz, Mt), 0)

        def rnd(i, carry):
            acc, bad = carry
            cv = cv_ref[...]
            mv = jnp.min(cv, axis=0, keepdims=True)
            sel = jnp.min(jnp.where(cv <= mv, sub5, Csz), axis=0,
                          keepdims=True)
            hit = sub5 == sel
            idxsel = jnp.min(jnp.where(hit, ci_ref[...],
                                       jnp.int32(0x7FFFFFFF)),
                             axis=0, keepdims=True)
            acc = jnp.where(krow == i, idxsel, acc)
            bad = jnp.maximum(bad, (sel % T == T - 1).astype(jnp.int32))
            cv_ref[...] = jnp.where(hit, INF, cv)
            return acc, bad

        acc, bad = lax.fori_loop(
            0, k, rnd,
            (jnp.zeros((k, Mt), jnp.int32), jnp.zeros((1, Mt), jnp.int32)))
        idx_ref[0] = acc + b * N

        @pl.when(jnp.max(bad) > 0)
        def _():
            idx_ref[0] = full_extract() + b * N

    return pl.pallas_call(
        body,
        grid=(B, M // Mt),
        in_specs=[
            pl.BlockSpec((1, N, 8), lambda b, m: (b, 0, 0)),
            pl.BlockSpec((1, 8, Mt), lambda b, m: (b, 0, m)),
        ],
        out_specs=pl.BlockSpec((1, k, Mt), lambda b, m: (b, 0, m)),
        out_shape=jax.ShapeDtypeStruct((B, k, M), jnp.int32),
        scratch_shapes=[
            pltpu.VMEM((N, Mt), _F32),
            pltpu.VMEM((max(Csz, 8), Mt), _F32),
            pltpu.VMEM((max(Csz, 8), Mt), jnp.int32),
        ],
    )(xyz_pad, centersT_pad)


def _sc_gather(table, idx):
    R, Dp = table.shape
    (Btot,) = idx.shape
    info = plsc.get_sparse_core_info()
    NW = info.num_cores * info.num_subcores
    CH = 128
    b_per_w = Btot // NW
    nch = b_per_w // CH
    mesh = plsc.VectorSubcoreMesh(core_axis_name="c", subcore_axis_name="s")

    nbuf = 2

    @functools.partial(
        pl.kernel,
        mesh=mesh,
        out_type=jax.ShapeDtypeStruct((Btot, Dp), _F32),
        scratch_types=[
            pltpu.VMEM((b_per_w,), jnp.int32),
            pltpu.VMEM((nbuf, CH, Dp), _F32),
            pltpu.SemaphoreType.DMA,
            pltpu.SemaphoreType.DMA,
        ],
    )
    def k(table_hbm, idx_hbm, out_hbm, idx_all, rows_v, sem0, sem1):
        wid = lax.axis_index("s") * info.num_cores + lax.axis_index("c")
        wbase = wid * b_per_w
        sems = (sem0, sem1)
        pltpu.sync_copy(idx_hbm.at[pl.ds(wbase, b_per_w)], idx_all)
        for b in range(nbuf):
            pltpu.async_copy(
                table_hbm.at[idx_all.at[pl.ds(b * CH, CH)]],
                rows_v.at[b], sems[b])

        def round_(r, carry):
            for b in range(nbuf):
                i = r * nbuf + b
                pltpu.make_async_copy(
                    table_hbm.at[idx_all.at[pl.ds(i * CH, CH)]],
                    rows_v.at[b], sems[b]).wait()
                pltpu.sync_copy(rows_v.at[b],
                                out_hbm.at[pl.ds(wbase + i * CH, CH)])

                @pl.when(i + nbuf < nch)
                def _():
                    pltpu.async_copy(
                        table_hbm.at[idx_all.at[pl.ds((i + nbuf) * CH, CH)]],
                        rows_v.at[b], sems[b])
            return carry

        lax.fori_loop(0, nch // nbuf, round_, 0)

    return k(table, idx)


def _sa_dense(gath, centers_pad, WfT, bf, WaT, ba, C, Mt):
    B, M, K_, Dp = gath.shape
    Cf = WfT.shape[1]
    Cin = 3 + C

    def body(g_ref, c_ref, wf_ref, bf_ref, wa_ref, ba_ref, o_ref):
        g = g_ref[0]
        cen = c_ref[0][:, :3]
        local = g[:, :, :3] - cen[:, None, :]
        cat = jnp.concatenate([local, g[:, :, 3:3 + C]], axis=2)
        x2 = cat.reshape(Mt * K_, Cin)
        f = jnp.maximum(
            jnp.dot(x2, wf_ref[...], preferred_element_type=_F32) + bf_ref[...], 0.0)
        f3 = f.reshape(Mt, K_, Cf)
        fm = jnp.mean(f3, axis=1, keepdims=True)
        ax = jnp.concatenate([cat, f3 - fm], axis=2).reshape(Mt * K_, Cin + Cf)
        a = jnp.maximum(
            jnp.dot(ax, wa_ref[...], preferred_element_type=_F32) + ba_ref[...], 0.0)
        a3 = a.reshape(Mt, K_, Cf)
        amax = jnp.max(a3, axis=1, keepdims=True)
        e = jnp.exp(a3 - amax)
        w = e / jnp.sum(e, axis=1, keepdims=True)
        o_ref[0] = jnp.sum(w * f3, axis=1)

    return pl.pallas_call(
        body,
        grid=(B, M // Mt),
        in_specs=[
            pl.BlockSpec((1, Mt, K_, Dp), lambda b, m: (b, m, 0, 0)),
            pl.BlockSpec((1, Mt, 8), lambda b, m: (b, m, 0)),
            pl.BlockSpec(WfT.shape, lambda b, m: (0, 0)),
            pl.BlockSpec(bf.shape, lambda b, m: (0, 0)),
            pl.BlockSpec(WaT.shape, lambda b, m: (0, 0)),
            pl.BlockSpec(ba.shape, lambda b, m: (0, 0)),
        ],
        out_specs=pl.BlockSpec((1, Mt, Cf), lambda b, m: (b, m, 0)),
        out_shape=jax.ShapeDtypeStruct((B, M, Cf), _F32),
    )(gath, centers_pad, WfT, bf, WaT, ba)


def _fp(xyzlT_pad, xyzh_pad, featlT, fhT, W, b, Mt, head=None):
    B = xyzlT_pad.shape[0]
    Nl = xyzlT_pad.shape[2]
    Nh = xyzh_pad.shape[1]
    Ch = fhT.shape[1]
    Cl = featlT.shape[1]
    Cout = W.shape[0]
    hw = head if head is not None else ()
    n_out = hw[2].shape[0] if head is not None else Cout

    def body(*refs):
        cT_ref, ph_ref, flT_ref, fhT_ref, w_ref, b_ref = refs[:6]
        o_ref = refs[-1]
        cT = cT_ref[0]
        p = ph_ref[0]
        cc = jnp.sum(cT * cT, axis=0, keepdims=True)
        pp = jnp.sum(p * p, axis=1, keepdims=True)
        d = jnp.sqrt(jnp.maximum(
            pp + cc - 2.0 * jnp.dot(p, cT, preferred_element_type=_F32), 0.0))
        sub = lax.broadcasted_iota(jnp.int32, (Nh, Mt), 0)
        sels, ws = [], []
        for _ in range(3):
            mv = jnp.min(d, axis=0, keepdims=True)
            sel = jnp.min(jnp.where(d <= mv, sub, Nh), axis=0, keepdims=True)
            ws.append(1.0 / jnp.maximum(mv, 1e-8))
            sels.append(sel)
            d = jnp.where(sub == sel, jnp.float32(jnp.inf), d)
        wsum = ws[0] + ws[1] + ws[2]
        WiT = ((ws[0] / wsum) * (sub == sels[0]).astype(_F32)
               + (ws[1] / wsum) * (sub == sels[1]).astype(_F32)
               + (ws[2] / wsum) * (sub == sels[2]).astype(_F32))
        fiT = jnp.dot(fhT_ref[0], WiT, preferred_element_type=_F32)
        x = jnp.concatenate([fiT, flT_ref[0]], axis=0)
        u = jnp.maximum(
            jnp.dot(w_ref[...], x, preferred_element_type=_F32) + b_ref[...], 0.0)
        if head is not None:
            h1w_ref, h1b_ref, h2w_ref, h2b_ref = refs[6:10]
            h = jnp.maximum(
                jnp.dot(h1w_ref[...], u, preferred_element_type=_F32)
                + h1b_ref[...], 0.0)
            u = (jnp.dot(h2w_ref[...], h, preferred_element_type=_F32)
                 + h2b_ref[...])
        o_ref[0] = u

    in_specs = [
        pl.BlockSpec((1, 8, Mt), lambda bb, m: (bb, 0, m)),
        pl.BlockSpec((1, Nh, 8), lambda bb, m: (bb, 0, 0)),
        pl.BlockSpec((1, Cl, Mt), lambda bb, m: (bb, 0, m)),
        pl.BlockSpec((1, Ch, Nh), lambda bb, m: (bb, 0, 0)),
        pl.BlockSpec(W.shape, lambda bb, m: (0, 0)),
        pl.BlockSpec(b.shape, lambda bb, m: (0, 0)),
    ]
    args = [xyzlT_pad, xyzh_pad, featlT, fhT, W, b]
    for w_ in hw:
        in_specs.append(pl.BlockSpec(w_.shape, lambda bb, m: (0, 0)))
        args.append(w_)

    return pl.pallas_call(
        body,
        grid=(B, Nl // Mt),
        in_specs=in_specs,
        out_specs=pl.BlockSpec((1, n_out, Mt), lambda bb, m: (bb, 0, m)),
        out_shape=jax.ShapeDtypeStruct((B, n_out, Nl), _F32),
    )(*args)


def _centers_idx(N, M):
    import numpy as np
    return jnp.asarray(np.linspace(0, N - 1, M).astype(np.int32))


def _sa_layer(xyz_pad, table, pf, pa, M, Mt_knn, Mt_dense):
    B, N, Dp = table.shape
    idxc = _centers_idx(N, M)
    centers_pad = xyz_pad[:, idxc, :]
    centersT_pad = jnp.transpose(centers_pad, (0, 2, 1))
    idx = _knn(centersT_pad, xyz_pad, _K, Mt_knn)
    idx = jnp.transpose(idx, (0, 2, 1))
    gath = _sc_gather(table.reshape(B * N, Dp), idx.reshape(-1))
    gath = gath.reshape(B, M, _K, Dp)
    WfT, bf = _fold(pf)
    WaT, ba = _fold(pa)
    Cfeat = WfT.shape[0] - 3
    f = _sa_dense(gath, centers_pad, WfT, bf, WaT, ba, Cfeat, Mt_dense)
    return centers_pad, f


def kernel(X, params):
    B, N, _ = X.shape
    xyz = X[:, :, :3]
    xyz_pad = _pad_cols(xyz, 8)

    WsT, bs = _fold(params['spfe'])
    W1, W2 = WsT[0:3], WsT[3:6]
    Wc = _pad_cols((W1 + W2).T, 8).T
    Wm = _pad_cols(W2.T, 8).T
    feat0 = _spfe(xyz_pad, Wc, Wm, bs)

    M1, M2, M3 = N // 4, N // 8, N // 16

    T1 = _pad_cols(jnp.concatenate([xyz, feat0], axis=-1), 128)
    c1_pad, f1 = _sa_layer(xyz_pad, T1, params['sa1_f'], params['sa1_a'],
                           M1, 1024, 128)
    xyz1 = c1_pad[:, :, :3]

    T2 = _pad_cols(jnp.concatenate([xyz1, f1, xyz1], axis=-1), 256)
    c2_pad, f2 = _sa_layer(c1_pad, T2, params['sa2_f'], params['sa2_a'],
                           M2, 1024, 128)
    xyz2 = c2_pad[:, :, :3]

    T3 = _pad_cols(jnp.concatenate([xyz2, f2, xyz2], axis=-1), 384)
    c3_pad, f3 = _sa_layer(c2_pad, T3, params['sa3_f'], params['sa3_a'],
                           M3, 512, 128)

    def fold_cf(p):
        s = p['g'] / jnp.sqrt(1.0 + _EPS)
        return p['W'] * s[:, None], (p['b'] * s + p['bb'])[:, None]

    c1T = jnp.transpose(c1_pad, (0, 2, 1))
    c2T = jnp.transpose(c2_pad, (0, 2, 1))
    f1T = jnp.transpose(f1, (0, 2, 1))
    f2T = jnp.transpose(f2, (0, 2, 1))
    f3T = jnp.transpose(f3, (0, 2, 1))
    feat0T = jnp.transpose(feat0, (0, 2, 1))
    xyzT_pad = jnp.transpose(xyz_pad, (0, 2, 1))

    W3, b3 = fold_cf(params['fp3'])
    u3 = _fp(c2T, c3_pad, f2T, f3T, W3, b3, 256)
    W2, b2 = fold_cf(params['fp2'])
    u2 = _fp(c1T, c2_pad, f1T, u3, W2, b2, 256)
    W1, b1 = fold_cf(params['fp1'])
    H1, h1b = fold_cf(params['head1'])
    H2 = params['head2']['W']
    h2b = params['head2']['b'][:, None]
    return _fp(xyzT_pad, c1_pad, feat0T, u2, W1, b1, 256,
               head=(H1, h1b, H2, h2b))


# speedup vs baseline: 1.1195x; 1.0521x over previous
import functools

import jax
import jax.numpy as jnp
from jax import lax
from jax.experimental import pallas as pl
from jax.experimental.pallas import tpu as pltpu
from jax.experimental.pallas import tpu_sc as plsc

_EPS = 1e-5
_K = 32
_F32 = jnp.float32


def _fold(p):
    s = p['g'] / jnp.sqrt(1.0 + _EPS)
    W = p['W'] * s[:, None]
    b = p['b'] * s + p['bb']
    return W.T, b[None, :]


def _pad_cols(x, to):
    c = x.shape[-1]
    if c == to:
        return x
    pad = [(0, 0)] * (x.ndim - 1) + [(0, to - c)]
    return jnp.pad(x, pad)


def _spfe(xyz_pad, Wc, Wm, b):
    B, N, _ = xyz_pad.shape
    Cout = Wc.shape[1]

    def body(x_ref, wc_ref, wm_ref, b_ref, o_ref):
        x = x_ref[0]
        m = jnp.mean(x, axis=0, keepdims=True)
        y = (jnp.dot(x, wc_ref[...], preferred_element_type=_F32)
             - jnp.dot(m, wm_ref[...], preferred_element_type=_F32)
             + b_ref[...])
        o_ref[0] = jnp.maximum(y, 0.0)

    return pl.pallas_call(
        body,
        grid=(B,),
        in_specs=[
            pl.BlockSpec((1, N, 8), lambda i: (i, 0, 0)),
            pl.BlockSpec((8, Cout), lambda i: (0, 0)),
            pl.BlockSpec((8, Cout), lambda i: (0, 0)),
            pl.BlockSpec((1, Cout), lambda i: (0, 0)),
        ],
        out_specs=pl.BlockSpec((1, N, Cout), lambda i: (i, 0, 0)),
        out_shape=jax.ShapeDtypeStruct((B, N, Cout), _F32),
    )(xyz_pad, Wc, Wm, b)


def _knn(centersT_pad, xyz_pad, k, Mt):
    B = centersT_pad.shape[0]
    M = centersT_pad.shape[2]
    N = xyz_pad.shape[1]

    cw = 128
    nc = N // cw
    T = 8 if nc >= 32 else 16
    chunked = nc >= 8
    Csz = nc * T
    INF = float('inf')

    def body(p_ref, c_ref, idx_ref, d_ref, cv_ref, ci_ref):
        b = pl.program_id(0)
        p = p_ref[0]
        cT = c_ref[0]
        pp = jnp.sum(p * p, axis=1, keepdims=True)
        cc = jnp.sum(cT * cT, axis=0, keepdims=True)
        d2 = pp + cc - 2.0 * jnp.dot(p, cT, preferred_element_type=_F32)
        d_ref[...] = jnp.sqrt(jnp.maximum(d2, 0.0))
        sub = lax.broadcasted_iota(jnp.int32, (N, Mt), 0)
        krow = lax.broadcasted_iota(jnp.int32, (k, Mt), 0)

        def full_extract():
            def it(i, acc):
                d_ = d_ref[...]
                mv = jnp.min(d_, axis=0, keepdims=True)
                sel = jnp.min(jnp.where(d_ <= mv, sub, N), axis=0,
                              keepdims=True)
                acc = jnp.where(krow == i, sel, acc)
                d_ref[...] = jnp.where(sub == sel, INF, d_)
                return acc

            return lax.fori_loop(0, k, it, jnp.zeros((k, Mt), jnp.int32))

        if not chunked:
            idx_ref[0] = full_extract() + b * N
            return

        subw = lax.broadcasted_iota(jnp.int32, (cw, Mt), 0)
        trow = lax.broadcasted_iota(jnp.int32, (T, Mt), 0)

        def per_chunk(c, carry):
            slab = d_ref[pl.ds(c * cw, cw), :]
            cand_v = jnp.full((T, Mt), INF, _F32)
            cand_i = jnp.zeros((T, Mt), jnp.int32)
            for t in range(T):
                mv = jnp.min(slab, axis=0, keepdims=True)
                sel = jnp.min(jnp.where(slab <= mv, subw, cw), axis=0,
                              keepdims=True)
                cand_v = jnp.where(trow == t, mv, cand_v)
                cand_i = jnp.where(trow == t, sel + c * cw, cand_i)
                slab = jnp.where(subw == sel, INF, slab)
            cv_ref[pl.ds(c * T, T), :] = cand_v
            ci_ref[pl.ds(c * T, T), :] = cand_i
            return carry

        lax.fori_loop(0, nc, per_chunk, 0)

        sub5 = lax.broadcasted_iota(jnp.int32, (Csz, Mt), 0)

        def rnd(i, carry):
            acc, bad = carry
            cv = cv_ref[...]
            mv = jnp.min(cv, axis=0, keepdims=True)
            sel = jnp.min(jnp.where(cv <= mv, sub5, Csz), axis=0,
                          keepdims=True)
            hit = sub5 == sel
            idxsel = jnp.min(jnp.where(hit, ci_ref[...],
                                       jnp.int32(0x7FFFFFFF)),
                             axis=0, keepdims=True)
            acc = jnp.where(krow == i, idxsel, acc)
            bad = jnp.maximum(bad, (sel % T == T - 1).astype(jnp.int32))
            cv_ref[...] = jnp.where(hit, INF, cv)
            return acc, bad

        acc, bad = lax.fori_loop(
            0, k, rnd,
            (jnp.zeros((k, Mt), jnp.int32), jnp.zeros((1, Mt), jnp.int32)))
        idx_ref[0] = acc + b * N

        @pl.when(jnp.max(bad) > 0)
        def _():
            idx_ref[0] = full_extract() + b * N

    return pl.pallas_call(
        body,
        grid=(B, M // Mt),
        in_specs=[
            pl.BlockSpec((1, N, 8), lambda b, m: (b, 0, 0)),
            pl.BlockSpec((1, 8, Mt), lambda b, m: (b, 0, m)),
        ],
        out_specs=pl.BlockSpec((1, k, Mt), lambda b, m: (b, 0, m)),
        out_shape=jax.ShapeDtypeStruct((B, k, M), jnp.int32),
        scratch_shapes=[
            pltpu.VMEM((N, Mt), _F32),
            pltpu.VMEM((max(Csz, 8), Mt), _F32),
            pltpu.VMEM((max(Csz, 8), Mt), jnp.int32),
        ],
    )(xyz_pad, centersT_pad)


def _sc_gather(table, idx):
    R, Dp = table.shape
    (Btot,) = idx.shape
    info = plsc.get_sparse_core_info()
    NW = info.num_cores * info.num_subcores
    CH = 128
    b_per_w = Btot // NW
    nch = b_per_w // CH
    mesh = plsc.VectorSubcoreMesh(core_axis_name="c", subcore_axis_name="s")

    nbuf = 2

    @functools.partial(
        pl.kernel,
        mesh=mesh,
        out_type=jax.ShapeDtypeStruct((Btot, Dp), _F32),
        scratch_types=[
            pltpu.VMEM((b_per_w,), jnp.int32),
            pltpu.VMEM((nbuf, CH, Dp), _F32),
            pltpu.SemaphoreType.DMA,
            pltpu.SemaphoreType.DMA,
        ],
    )
    def k(table_hbm, idx_hbm, out_hbm, idx_all, rows_v, sem0, sem1):
        wid = lax.axis_index("s") * info.num_cores + lax.axis_index("c")
        wbase = wid * b_per_w
        sems = (sem0, sem1)
        pltpu.sync_copy(idx_hbm.at[pl.ds(wbase, b_per_w)], idx_all)
        for b in range(nbuf):
            pltpu.async_copy(
                table_hbm.at[idx_all.at[pl.ds(b * CH, CH)]],
                rows_v.at[b], sems[b])

        def round_(r, carry):
            for b in range(nbuf):
                i = r * nbuf + b
                pltpu.make_async_copy(
                    table_hbm.at[idx_all.at[pl.ds(i * CH, CH)]],
                    rows_v.at[b], sems[b]).wait()
                pltpu.sync_copy(rows_v.at[b],
                                out_hbm.at[pl.ds(wbase + i * CH, CH)])

                @pl.when(i + nbuf < nch)
                def _():
                    pltpu.async_copy(
                        table_hbm.at[idx_all.at[pl.ds((i + nbuf) * CH, CH)]],
                        rows_v.at[b], sems[b])
            return carry

        lax.fori_loop(0, nch // nbuf, round_, 0)

    return k(table, idx)


def _sa_dense(gath, centers_pad, WfT, bf, WaT, ba, C, Mt):
    B, M, K_, Dp = gath.shape
    Cf = WfT.shape[1]
    Cin = 3 + C

    def body(g_ref, c_ref, wf_ref, bf_ref, wa_ref, ba_ref, o_ref):
        g = g_ref[0]
        cen = c_ref[0][:, :3]
        local = g[:, :, :3] - cen[:, None, :]
        cat = jnp.concatenate([local, g[:, :, 3:3 + C]], axis=2)
        x2 = cat.reshape(Mt * K_, Cin)
        f = jnp.maximum(
            jnp.dot(x2, wf_ref[...], preferred_element_type=_F32) + bf_ref[...], 0.0)
        f3 = f.reshape(Mt, K_, Cf)
        fm = jnp.mean(f3, axis=1, keepdims=True)
        ax = jnp.concatenate([cat, f3 - fm], axis=2).reshape(Mt * K_, Cin + Cf)
        a = jnp.maximum(
            jnp.dot(ax, wa_ref[...], preferred_element_type=_F32) + ba_ref[...], 0.0)
        a3 = a.reshape(Mt, K_, Cf)
        amax = jnp.max(a3, axis=1, keepdims=True)
        e = jnp.exp(a3 - amax)
        w = e / jnp.sum(e, axis=1, keepdims=True)
        o_ref[0] = jnp.sum(w * f3, axis=1)

    return pl.pallas_call(
        body,
        grid=(B, M // Mt),
        in_specs=[
            pl.BlockSpec((1, Mt, K_, Dp), lambda b, m: (b, m, 0, 0)),
            pl.BlockSpec((1, Mt, 8), lambda b, m: (b, m, 0)),
            pl.BlockSpec(WfT.shape, lambda b, m: (0, 0)),
            pl.BlockSpec(bf.shape, lambda b, m: (0, 0)),
            pl.BlockSpec(WaT.shape, lambda b, m: (0, 0)),
            pl.BlockSpec(ba.shape, lambda b, m: (0, 0)),
        ],
        out_specs=pl.BlockSpec((1, Mt, Cf), lambda b, m: (b, m, 0)),
        out_shape=jax.ShapeDtypeStruct((B, M, Cf), _F32),
    )(gath, centers_pad, WfT, bf, WaT, ba)


def _fp(xyzlT_pad, xyzh_pad, featlT, fhT, W, b, Mt, head=None):
    B = xyzlT_pad.shape[0]
    Nl = xyzlT_pad.shape[2]
    Nh = xyzh_pad.shape[1]
    Ch = fhT.shape[1]
    Cl = featlT.shape[1]
    Cout = W.shape[0]
    hw = head if head is not None else ()
    n_out = hw[2].shape[0] if head is not None else Cout

    def body(*refs):
        cT_ref, ph_ref, flT_ref, fhT_ref, w_ref, b_ref = refs[:6]
        o_ref = refs[-1]
        cT = cT_ref[0]
        p = ph_ref[0]
        cc = jnp.sum(cT * cT, axis=0, keepdims=True)
        pp = jnp.sum(p * p, axis=1, keepdims=True)
        d = jnp.sqrt(jnp.maximum(
            pp + cc - 2.0 * jnp.dot(p, cT, preferred_element_type=_F32), 0.0))
        sub = lax.broadcasted_iota(jnp.int32, (Nh, Mt), 0)
        sels, ws = [], []
        for _ in range(3):
            mv = jnp.min(d, axis=0, keepdims=True)
            sel = jnp.min(jnp.where(d <= mv, sub, Nh), axis=0, keepdims=True)
            ws.append(1.0 / jnp.maximum(mv, 1e-8))
            sels.append(sel)
            d = jnp.where(sub == sel, jnp.float32(jnp.inf), d)
        wsum = ws[0] + ws[1] + ws[2]
        WiT = ((ws[0] / wsum) * (sub == sels[0]).astype(_F32)
               + (ws[1] / wsum) * (sub == sels[1]).astype(_F32)
               + (ws[2] / wsum) * (sub == sels[2]).astype(_F32))
        fiT = jnp.dot(fhT_ref[0], WiT, preferred_element_type=_F32)
        x = jnp.concatenate([fiT, flT_ref[0]], axis=0)
        u = jnp.maximum(
            jnp.dot(w_ref[...], x, preferred_element_type=_F32) + b_ref[...], 0.0)
        if head is not None:
            h1w_ref, h1b_ref, h2w_ref, h2b_ref = refs[6:10]
            h = jnp.maximum(
                jnp.dot(h1w_ref[...], u, preferred_element_type=_F32)
                + h1b_ref[...], 0.0)
            u = (jnp.dot(h2w_ref[...], h, preferred_element_type=_F32)
                 + h2b_ref[...])
        o_ref[0] = u

    in_specs = [
        pl.BlockSpec((1, 8, Mt), lambda bb, m: (bb, 0, m)),
        pl.BlockSpec((1, Nh, 8), lambda bb, m: (bb, 0, 0)),
        pl.BlockSpec((1, Cl, Mt), lambda bb, m: (bb, 0, m)),
        pl.BlockSpec((1, Ch, Nh), lambda bb, m: (bb, 0, 0)),
        pl.BlockSpec(W.shape, lambda bb, m: (0, 0)),
        pl.BlockSpec(b.shape, lambda bb, m: (0, 0)),
    ]
    args = [xyzlT_pad, xyzh_pad, featlT, fhT, W, b]
    for w_ in hw:
        in_specs.append(pl.BlockSpec(w_.shape, lambda bb, m: (0, 0)))
        args.append(w_)

    return pl.pallas_call(
        body,
        grid=(B, Nl // Mt),
        in_specs=in_specs,
        out_specs=pl.BlockSpec((1, n_out, Mt), lambda bb, m: (bb, 0, m)),
        out_shape=jax.ShapeDtypeStruct((B, n_out, Nl), _F32),
    )(*args)


def _centers_idx(N, M):
    import numpy as np
    return jnp.asarray(np.linspace(0, N - 1, M).astype(np.int32))


def _sa_layer(xyz_pad, table, pf, pa, M, Mt_knn, Mt_dense):
    B, N, Dp = table.shape
    idxc = _centers_idx(N, M)
    centers_pad = xyz_pad[:, idxc, :]
    centersT_pad = jnp.transpose(centers_pad, (0, 2, 1))
    idx = _knn(centersT_pad, xyz_pad, _K, Mt_knn)
    idx = jnp.transpose(idx, (0, 2, 1))
    gath = _sc_gather(table.reshape(B * N, Dp), idx.reshape(-1))
    gath = gath.reshape(B, M, _K, Dp)
    WfT, bf = _fold(pf)
    WaT, ba = _fold(pa)
    Cfeat = WfT.shape[0] - 3
    f = _sa_dense(gath, centers_pad, WfT, bf, WaT, ba, Cfeat, Mt_dense)
    return centers_pad, f


def kernel(X, params):
    B, N, _ = X.shape
    xyz = X[:, :, :3]
    xyz_pad = _pad_cols(xyz, 8)

    WsT, bs = _fold(params['spfe'])
    W1, W2 = WsT[0:3], WsT[3:6]
    Wc = _pad_cols((W1 + W2).T, 8).T
    Wm = _pad_cols(W2.T, 8).T
    feat0 = _spfe(xyz_pad, Wc, Wm, bs)

    M1, M2, M3 = N // 4, N // 8, N // 16

    T1 = _pad_cols(jnp.concatenate([xyz, feat0], axis=-1), 128)
    c1_pad, f1 = _sa_layer(xyz_pad, T1, params['sa1_f'], params['sa1_a'],
                           M1, 1024, 256)
    xyz1 = c1_pad[:, :, :3]

    T2 = _pad_cols(jnp.concatenate([xyz1, f1, xyz1], axis=-1), 256)
    c2_pad, f2 = _sa_layer(c1_pad, T2, params['sa2_f'], params['sa2_a'],
                           M2, 1024, 256)
    xyz2 = c2_pad[:, :, :3]

    T3 = _pad_cols(jnp.concatenate([xyz2, f2, xyz2], axis=-1), 384)
    c3_pad, f3 = _sa_layer(c2_pad, T3, params['sa3_f'], params['sa3_a'],
                           M3, 512, 128)

    def fold_cf(p):
        s = p['g'] / jnp.sqrt(1.0 + _EPS)
        return p['W'] * s[:, None], (p['b'] * s + p['bb'])[:, None]

    c1T = jnp.transpose(c1_pad, (0, 2, 1))
    c2T = jnp.transpose(c2_pad, (0, 2, 1))
    f1T = jnp.transpose(f1, (0, 2, 1))
    f2T = jnp.transpose(f2, (0, 2, 1))
    f3T = jnp.transpose(f3, (0, 2, 1))
    feat0T = jnp.transpose(feat0, (0, 2, 1))
    xyzT_pad = jnp.transpose(xyz_pad, (0, 2, 1))

    W3, b3 = fold_cf(params['fp3'])
    u3 = _fp(c2T, c3_pad, f2T, f3T, W3, b3, 512)
    W2, b2 = fold_cf(params['fp2'])
    u2 = _fp(c1T, c2_pad, f1T, u3, W2, b2, 512)
    W1, b1 = fold_cf(params['fp1'])
    H1, h1b = fold_cf(params['head1'])
    H2 = params['head2']['W']
    h2b = params['head2']['b'][:, None]
    return _fp(xyzT_pad, c1_pad, feat0T, u2, W1, b1, 512,
               head=(H1, h1b, H2, h2b))
